# Initial kernel scaffold; baseline (speedup 1.0000x reference)
import jax
import jax.numpy as jnp
from jax.experimental import pallas as pl


def kernel(xyz_low, xyz_high, points_low, points_high, W, b, gamma, beta, running_mean, running_var):
    raise NotImplementedError("write your pallas kernel here")


# speedup vs baseline: 28.0125x; 28.0125x over previous
import functools

import jax
import jax.numpy as jnp
import numpy as np
from jax.experimental import pallas as pl
from jax.experimental.pallas import tpu as pltpu

B, N, S = 2, 8192, 2048
LOW, HIGH = 512, 256
NT = 256


def _tile_kernel(xl_ref, xh_ref, x_ref, ph_ref, w_ref, bias_ref, out_ref, pl_scratch):
    n_idx = pl.program_id(1)

    @pl.when(n_idx == 0)
    def _compute_mlp():
        acc = jnp.dot(w_ref[...], x_ref[0], preferred_element_type=jnp.float32)
        pl_scratch[...] = jnp.maximum(acc + bias_ref[...], 0.0)

    xl = xl_ref[0]
    xh = xh_ref[0]
    dx = xl[:, 0:1] - xh[0:1, :]
    dy = xl[:, 1:2] - xh[1:2, :]
    dz = xl[:, 2:3] - xh[2:3, :]
    d2 = dx * dx + dy * dy + dz * dz

    iota = jax.lax.broadcasted_iota(jnp.int32, (S, NT), 0)
    d = d2
    mins = []
    idxs = []
    for _ in range(3):
        mk = jnp.min(d, axis=0, keepdims=True)
        ik = jnp.min(jnp.where(d == mk, iota, S), axis=0, keepdims=True)
        mins.append(mk)
        idxs.append(ik)
        d = jnp.where(iota == ik, jnp.float32(np.inf), d)

    r0 = 1.0 / (mins[0] + 1e-8)
    r1 = 1.0 / (mins[1] + 1e-8)
    r2 = 1.0 / (mins[2] + 1e-8)
    norm = r0 + r1 + r2
    w0 = r0 / norm
    w1 = r1 / norm
    w2 = r2 / norm

    zero = jnp.zeros((S, NT), jnp.float32)
    wsp = jnp.where(iota == idxs[0], w0, zero)
    wsp = wsp + jnp.where(iota == idxs[1], w1, zero)
    wsp = wsp + jnp.where(iota == idxs[2], w2, zero)

    interp = jnp.dot(pl_scratch[...], wsp, preferred_element_type=jnp.float32)
    out_ref[0] = interp + ph_ref[0]


@jax.jit
def kernel(xyz_low, xyz_high, points_low, points_high, W, b, gamma, beta,
           running_mean, running_var):
    scale = gamma / jnp.sqrt(running_var + 1e-5)
    w_folded = W * scale[:, None]
    b_folded = ((b - running_mean) * scale + beta)[:, None]
    xl_t = jnp.transpose(xyz_low, (0, 2, 1))

    grid = (B, N // NT)
    out = pl.pallas_call(
        _tile_kernel,
        grid=grid,
        in_specs=[
            pl.BlockSpec((1, S, 3), lambda bi, ni: (bi, 0, 0)),
            pl.BlockSpec((1, 3, NT), lambda bi, ni: (bi, 0, ni)),
            pl.BlockSpec((1, LOW, S), lambda bi, ni: (bi, 0, 0)),
            pl.BlockSpec((1, HIGH, NT), lambda bi, ni: (bi, 0, ni)),
            pl.BlockSpec((HIGH, LOW), lambda bi, ni: (0, 0)),
            pl.BlockSpec((HIGH, 1), lambda bi, ni: (0, 0)),
        ],
        out_specs=pl.BlockSpec((1, HIGH, NT), lambda bi, ni: (bi, 0, ni)),
        out_shape=jax.ShapeDtypeStruct((B, HIGH, N), jnp.float32),
        scratch_shapes=[pltpu.VMEM((HIGH, S), jnp.float32)],
    )(xl_t, xyz_high, points_low, points_high, w_folded, b_folded)
    return out


# speedup vs baseline: 29.2805x; 1.0453x over previous
import functools

import jax
import jax.numpy as jnp
import numpy as np
from jax.experimental import pallas as pl
from jax.experimental.pallas import tpu as pltpu

B, N, S = 2, 8192, 2048
LOW, HIGH = 512, 256
NT = 256


def _tile_kernel(xl_ref, xh_ref, x_ref, ph_ref, w_ref, bias_ref, out_ref,
                 pl_scratch, nl_scratch):
    n_idx = pl.program_id(1)

    @pl.when(n_idx == 0)
    def _per_batch():
        acc = jnp.dot(w_ref[...], x_ref[0], preferred_element_type=jnp.float32)
        pl_scratch[...] = jnp.maximum(acc + bias_ref[...], 0.0)
        xl0 = xl_ref[0]
        nl_scratch[...] = (xl0[:, 0:1] * xl0[:, 0:1]
                           + xl0[:, 1:2] * xl0[:, 1:2]
                           + xl0[:, 2:3] * xl0[:, 2:3])

    xl = xl_ref[0]
    xh = xh_ref[0]
    g = jnp.dot(xl, xh, preferred_element_type=jnp.float32,
                precision=jax.lax.Precision.HIGHEST)
    e = nl_scratch[...] - 2.0 * g

    inf = jnp.float32(np.inf)
    m0 = jnp.min(e, axis=0, keepdims=True)
    d1 = jnp.where(e > m0, e, inf)
    m1 = jnp.min(d1, axis=0, keepdims=True)
    d2 = jnp.where(d1 > m1, d1, inf)
    m2 = jnp.min(d2, axis=0, keepdims=True)

    nh = (xh[0:1, :] * xh[0:1, :] + xh[1:2, :] * xh[1:2, :]
          + xh[2:3, :] * xh[2:3, :])
    r0 = 1.0 / (jnp.maximum(m0 + nh, 0.0) + 1e-8)
    r1 = 1.0 / (jnp.maximum(m1 + nh, 0.0) + 1e-8)
    r2 = 1.0 / (jnp.maximum(m2 + nh, 0.0) + 1e-8)
    norm = r0 + r1 + r2
    w0 = r0 / norm
    w1 = r1 / norm
    w2 = r2 / norm

    zero = jnp.float32(0.0)
    wsp = jnp.where(e == m0, w0,
                    jnp.where(e == m1, w1,
                              jnp.where(e == m2, w2, zero)))

    interp = jnp.dot(pl_scratch[...], wsp, preferred_element_type=jnp.float32)
    out_ref[0] = interp + ph_ref[0]


@jax.jit
def kernel(xyz_low, xyz_high, points_low, points_high, W, b, gamma, beta,
           running_mean, running_var):
    scale = gamma / jnp.sqrt(running_var + 1e-5)
    w_folded = W * scale[:, None]
    b_folded = ((b - running_mean) * scale + beta)[:, None]
    xl_t = jnp.transpose(xyz_low, (0, 2, 1))

    grid = (B, N // NT)
    out = pl.pallas_call(
        _tile_kernel,
        grid=grid,
        in_specs=[
            pl.BlockSpec((1, S, 3), lambda bi, ni: (bi, 0, 0)),
            pl.BlockSpec((1, 3, NT), lambda bi, ni: (bi, 0, ni)),
            pl.BlockSpec((1, LOW, S), lambda bi, ni: (bi, 0, 0)),
            pl.BlockSpec((1, HIGH, NT), lambda bi, ni: (bi, 0, ni)),
            pl.BlockSpec((HIGH, LOW), lambda bi, ni: (0, 0)),
            pl.BlockSpec((HIGH, 1), lambda bi, ni: (0, 0)),
        ],
        out_specs=pl.BlockSpec((1, HIGH, NT), lambda bi, ni: (bi, 0, ni)),
        out_shape=jax.ShapeDtypeStruct((B, HIGH, N), jnp.float32),
        scratch_shapes=[
            pltpu.VMEM((HIGH, S), jnp.float32),
            pltpu.VMEM((S, 1), jnp.float32),
        ],
    )(xl_t, xyz_high, points_low, points_high, w_folded, b_folded)
    return out


# speedup vs baseline: 37.6178x; 1.2847x over previous
import functools

import jax
import jax.numpy as jnp
import numpy as np
from jax.experimental import pallas as pl
from jax.experimental.pallas import tpu as pltpu

B, N, S = 2, 8192, 2048
LOW, HIGH = 512, 256
NT = 256


def _tile_kernel(xl_ref, xh_ref, x_ref, ph_ref, w_ref, bias_ref, out_ref,
                 pl_scratch, nl_scratch):
    n_idx = pl.program_id(1)

    @pl.when(n_idx == 0)
    def _per_batch():
        acc = jnp.dot(w_ref[...], x_ref[0], preferred_element_type=jnp.float32)
        pl_scratch[...] = jnp.maximum(acc + bias_ref[...], 0.0)
        xl0 = xl_ref[0]
        nl_scratch[...] = 0.25 * (xl0[:, 0:1] * xl0[:, 0:1]
                                  + xl0[:, 1:2] * xl0[:, 1:2]
                                  + xl0[:, 2:3] * xl0[:, 2:3])

    xl = xl_ref[0]
    xh = xh_ref[0]
    e = (nl_scratch[...]
         + xl[:, 0:1] * xh[0:1, :]
         + xl[:, 1:2] * xh[1:2, :]
         + xl[:, 2:3] * xh[2:3, :])

    inf = jnp.float32(np.inf)
    m0 = jnp.min(e, axis=0, keepdims=True)
    d1 = jnp.where(e > m0, e, inf)
    m1 = jnp.min(d1, axis=0, keepdims=True)
    d2 = jnp.where(d1 > m1, d1, inf)
    m2 = jnp.min(d2, axis=0, keepdims=True)

    nh = (xh[0:1, :] * xh[0:1, :] + xh[1:2, :] * xh[1:2, :]
          + xh[2:3, :] * xh[2:3, :])
    r0 = 1.0 / (jnp.maximum(m0 + nh, 0.0) + 1e-8)
    r1 = 1.0 / (jnp.maximum(m1 + nh, 0.0) + 1e-8)
    r2 = 1.0 / (jnp.maximum(m2 + nh, 0.0) + 1e-8)
    norm = r0 + r1 + r2
    w0 = r0 / norm
    w1 = r1 / norm
    w2 = r2 / norm

    zero = jnp.float32(0.0)
    wsp = jnp.where(e == m0, w0,
                    jnp.where(e == m1, w1,
                              jnp.where(e == m2, w2, zero)))

    interp = jnp.dot(pl_scratch[...], wsp, preferred_element_type=jnp.float32)
    out_ref[0] = interp + ph_ref[0]


@jax.jit
def kernel(xyz_low, xyz_high, points_low, points_high, W, b, gamma, beta,
           running_mean, running_var):
    scale = gamma / jnp.sqrt(running_var + 1e-5)
    w_folded = W * scale[:, None]
    b_folded = ((b - running_mean) * scale + beta)[:, None]
    xl_t = -2.0 * jnp.transpose(xyz_low, (0, 2, 1))

    grid = (B, N // NT)
    out = pl.pallas_call(
        _tile_kernel,
        grid=grid,
        in_specs=[
            pl.BlockSpec((1, S, 3), lambda bi, ni: (bi, 0, 0)),
            pl.BlockSpec((1, 3, NT), lambda bi, ni: (bi, 0, ni)),
            pl.BlockSpec((1, LOW, S), lambda bi, ni: (bi, 0, 0)),
            pl.BlockSpec((1, HIGH, NT), lambda bi, ni: (bi, 0, ni)),
            pl.BlockSpec((HIGH, LOW), lambda bi, ni: (0, 0)),
            pl.BlockSpec((HIGH, 1), lambda bi, ni: (0, 0)),
        ],
        out_specs=pl.BlockSpec((1, HIGH, NT), lambda bi, ni: (bi, 0, ni)),
        out_shape=jax.ShapeDtypeStruct((B, HIGH, N), jnp.float32),
        scratch_shapes=[
            pltpu.VMEM((HIGH, S), jnp.float32),
            pltpu.VMEM((S, 1), jnp.float32),
        ],
    )(xl_t, xyz_high, points_low, points_high, w_folded, b_folded)
    return out
